# baseline (speedup 1.0000x reference)
import functools

import jax
import jax.numpy as jnp
from jax import lax
from jax.experimental import pallas as pl
from jax.experimental.pallas import tpu as pltpu
from jax.experimental.pallas import tpu_sc as plsc

K = 8
S = 8192
C = 128
B = 64
BT = 12
BS = B - BT
L = 16
QROWS = 2048
NQ = S // QROWS
CH = 16
NCHUNK = QROWS // CH
NW = 32
TASKS_PER_W = (BS * (C // L)) // NW
NG = TASKS_PER_W * NQ
NEG = float("-inf")


def _tc_body(x_ref, o_ref):
    x = x_ref[0]
    iota = lax.broadcasted_iota(jnp.int32, (S, C), 0)
    neg = jnp.float32(NEG)
    for r in range(K):
        m = jnp.max(x, axis=0)
        o_ref[0, r, :] = m
        idx = jnp.min(jnp.where(x == m[None, :], iota, jnp.int32(S)), axis=0)
        x = jnp.where(iota == idx[None, :], neg, x)


def _topk_tc(inputs):
    return pl.pallas_call(
        _tc_body,
        grid=(BT,),
        in_specs=[pl.BlockSpec((1, S, C), lambda i: (i, 0, 0))],
        out_specs=pl.BlockSpec((1, K, C), lambda i: (i, 0, 0)),
        out_shape=jax.ShapeDtypeStruct((BT, K, C), jnp.float32),
    )(inputs)


def _bcast_bool(pred):
    return jnp.broadcast_to(pred, (L,))


def _insert_val(t, v):
    t = list(t)
    for l in range(K):
        gt = v > t[l]
        t[l], v = jnp.where(gt, v, t[l]), jnp.where(gt, t[l], v)
    return t


def _task_coords(g, wid):
    q = lax.rem(g, NQ)
    t = lax.div(g, NQ)
    tid = wid * TASKS_PER_W + t
    b = lax.div(tid, C // L)
    cb = lax.rem(tid, C // L)
    return b, cb, q


def _issue(x_hbm, buf, sem, g, wid):
    @pl.when(g < NG)
    def _():
        b, cb, q = _task_coords(g, wid)
        pltpu.async_copy(
            x_hbm.at[BT + b, pl.ds(q * QROWS, QROWS), pl.ds(cb * L, L)],
            buf, sem)


def _process(x_hbm, out_hbm, buf, cm, outb, sem, g, wid, tops):
    b, cb, q = _task_coords(g, wid)
    pltpu.make_async_copy(
        x_hbm.at[BT + b, pl.ds(q * QROWS, QROWS), pl.ds(cb * L, L)], buf,
        sem).wait()

    def p1_body(c, _):
        base = c * CH
        v = [buf[base + j] for j in range(CH)]
        while len(v) > 1:
            v = [jnp.maximum(v[2 * i], v[2 * i + 1])
                 for i in range(len(v) // 2)]
        cm[c] = v[0]
        return 0

    lax.fori_loop(0, NCHUNK, p1_body, 0, unroll=2)

    def p2_body(j, car):
        kv = list(car)
        kb = lax.bitcast_convert_type(cm[j], jnp.int32)
        mono = kb ^ ((kb >> 31) & jnp.int32(0x7FFFFFFF))
        key = (mono & jnp.int32(~(NCHUNK - 1))) | j
        for l in range(K):
            kv[l], key = jnp.maximum(kv[l], key), jnp.minimum(kv[l], key)
        return tuple(kv)

    init = tuple(jnp.full((L,), jnp.int32(-2**31), jnp.int32)
                 for _ in range(K))
    sel = lax.fori_loop(0, NCHUNK, p2_body, init, unroll=8)
    chunk_idx = [kv & jnp.int32(NCHUNK - 1) for kv in sel]

    fresh = _bcast_bool(q == 0)
    tops = [jnp.where(fresh, jnp.float32(NEG), tv) for tv in tops]

    lane = lax.broadcasted_iota(jnp.int32, (L,), 0)
    for r in range(K):
        base = chunk_idx[r] * CH

        def p3_body(j, car, base=base):
            rows = base + j
            gv = plsc.load_gather(buf, [rows, lane])
            return tuple(_insert_val(list(car), gv))

        tops = list(lax.fori_loop(0, CH, p3_body, tuple(tops), unroll=8))

    @pl.when(q == NQ - 1)
    def _():
        for r in range(K):
            outb[r] = tops[r]
        pltpu.sync_copy(outb, out_hbm.at[b, :, pl.ds(cb * L, L)])

    return tops


def _topk_sc(inputs):
    mesh = plsc.VectorSubcoreMesh(core_axis_name="c", subcore_axis_name="s")

    @functools.partial(
        pl.kernel,
        mesh=mesh,
        out_type=jax.ShapeDtypeStruct((BS, K, C), jnp.float32),
        compiler_params=pltpu.CompilerParams(use_tc_tiling_on_sc=False,
                                             needs_layout_passes=False),
        scratch_types=[
            pltpu.VMEM((QROWS, L), jnp.float32),
            pltpu.VMEM((QROWS, L), jnp.float32),
            pltpu.VMEM((NCHUNK, L), jnp.float32),
            pltpu.VMEM((K, L), jnp.float32),
            pltpu.SemaphoreType.DMA,
            pltpu.SemaphoreType.DMA,
        ],
    )
    def run(x_hbm, out_hbm, buf_a, buf_b, cm, outb, sem_a, sem_b):
        wid = lax.axis_index("s") * 2 + lax.axis_index("c")
        _issue(x_hbm, buf_a, sem_a, jnp.int32(0), wid)
        _issue(x_hbm, buf_b, sem_b, jnp.int32(1), wid)

        def body(i, tops):
            g0 = i * 2
            tops = _process(x_hbm, out_hbm, buf_a, cm, outb, sem_a, g0, wid,
                            list(tops))
            _issue(x_hbm, buf_a, sem_a, g0 + 2, wid)
            tops = _process(x_hbm, out_hbm, buf_b, cm, outb, sem_b, g0 + 1,
                            wid, tops)
            _issue(x_hbm, buf_b, sem_b, g0 + 3, wid)
            return tuple(tops)

        init = tuple(jnp.full((L,), NEG, jnp.float32) for _ in range(K))
        lax.fori_loop(0, NG // 2, body, init)

    return run(inputs)


def kernel(inputs):
    out_tc = _topk_tc(inputs)
    out_sc = _topk_sc(inputs)
    return jnp.concatenate([out_tc, out_sc], axis=0)


# speedup vs baseline: 1.0338x; 1.0338x over previous
import functools

import jax
import jax.numpy as jnp
from jax import lax
from jax.experimental import pallas as pl
from jax.experimental.pallas import tpu as pltpu
from jax.experimental.pallas import tpu_sc as plsc

K = 8
S = 8192
C = 128
B = 64
BT = 12
BS = B - BT
L = 16
QROWS = 2048
NQ = S // QROWS
CH = 16
NCHUNK = QROWS // CH
NW = 32
TASKS_PER_W = (BS * (C // L)) // NW
NG = TASKS_PER_W * NQ
NEG = float("-inf")


def _tc_body(x_ref, o_ref):
    x = x_ref[0]
    iota = lax.broadcasted_iota(jnp.int32, (S, C), 0)
    neg = jnp.float32(NEG)
    for r in range(K):
        m = jnp.max(x, axis=0)
        o_ref[0, r, :] = m
        idx = jnp.min(jnp.where(x == m[None, :], iota, jnp.int32(S)), axis=0)
        x = jnp.where(iota == idx[None, :], neg, x)


def _topk_tc(inputs):
    return pl.pallas_call(
        _tc_body,
        grid=(BT,),
        in_specs=[pl.BlockSpec((1, S, C), lambda i: (i, 0, 0))],
        out_specs=pl.BlockSpec((1, K, C), lambda i: (i, 0, 0)),
        out_shape=jax.ShapeDtypeStruct((BT, K, C), jnp.float32),
    )(inputs)


def _bcast_bool(pred):
    return jnp.broadcast_to(pred, (L,))


def _insert_val(t, v):
    t = list(t)
    for l in range(K):
        gt = v > t[l]
        t[l], v = jnp.where(gt, v, t[l]), jnp.where(gt, t[l], v)
    return t


def _task_coords(g, wid):
    q = lax.rem(g, NQ)
    t = lax.div(g, NQ)
    tid = wid * TASKS_PER_W + t
    b = lax.div(tid, C // L)
    cb = lax.rem(tid, C // L)
    return b, cb, q


def _issue(x_hbm, buf, sem, g, wid):
    @pl.when(g < NG)
    def _():
        b, cb, q = _task_coords(g, wid)
        pltpu.async_copy(
            x_hbm.at[BT + b, pl.ds(q * QROWS, QROWS), pl.ds(cb * L, L)],
            buf, sem)


def _process(x_hbm, out_hbm, buf, cm, outb, sem, g, wid, tops):
    b, cb, q = _task_coords(g, wid)
    pltpu.make_async_copy(
        x_hbm.at[BT + b, pl.ds(q * QROWS, QROWS), pl.ds(cb * L, L)], buf,
        sem).wait()

    def p1_body(c, _):
        base = c * CH
        v = [buf[base + j] for j in range(CH)]
        while len(v) > 1:
            v = [jnp.maximum(v[2 * i], v[2 * i + 1])
                 for i in range(len(v) // 2)]
        cm[c] = v[0]
        return 0

    lax.fori_loop(0, NCHUNK, p1_body, 0)

    def p2_body(j, car):
        kv = list(car)
        kb = lax.bitcast_convert_type(cm[j], jnp.int32)
        mono = kb ^ ((kb >> 31) & jnp.int32(0x7FFFFFFF))
        key = (mono & jnp.int32(~(NCHUNK - 1))) | j
        for l in range(K):
            kv[l], key = jnp.maximum(kv[l], key), jnp.minimum(kv[l], key)
        return tuple(kv)

    init = tuple(jnp.full((L,), jnp.int32(-2**31), jnp.int32)
                 for _ in range(K))
    sel = lax.fori_loop(0, NCHUNK, p2_body, init, unroll=4)
    chunk_idx = [kv & jnp.int32(NCHUNK - 1) for kv in sel]

    fresh = _bcast_bool(q == 0)
    tops = [jnp.where(fresh, jnp.float32(NEG), tv) for tv in tops]

    lane = lax.broadcasted_iota(jnp.int32, (L,), 0)
    for r in range(K):
        base = chunk_idx[r] * CH

        def p3_body(j, car, base=base):
            rows = base + j
            gv = plsc.load_gather(buf, [rows, lane])
            return tuple(_insert_val(list(car), gv))

        tops = list(lax.fori_loop(0, CH, p3_body, tuple(tops), unroll=4))

    @pl.when(q == NQ - 1)
    def _():
        for r in range(K):
            outb[r] = tops[r]
        pltpu.sync_copy(outb, out_hbm.at[b, :, pl.ds(cb * L, L)])

    return tops


def _topk_sc(inputs):
    mesh = plsc.VectorSubcoreMesh(core_axis_name="c", subcore_axis_name="s")

    @functools.partial(
        pl.kernel,
        mesh=mesh,
        out_type=jax.ShapeDtypeStruct((BS, K, C), jnp.float32),
        compiler_params=pltpu.CompilerParams(use_tc_tiling_on_sc=False,
                                             needs_layout_passes=False),
        scratch_types=[
            pltpu.VMEM((QROWS, L), jnp.float32),
            pltpu.VMEM((QROWS, L), jnp.float32),
            pltpu.VMEM((NCHUNK, L), jnp.float32),
            pltpu.VMEM((K, L), jnp.float32),
            pltpu.SemaphoreType.DMA,
            pltpu.SemaphoreType.DMA,
        ],
    )
    def run(x_hbm, out_hbm, buf_a, buf_b, cm, outb, sem_a, sem_b):
        wid = lax.axis_index("s") * 2 + lax.axis_index("c")
        _issue(x_hbm, buf_a, sem_a, jnp.int32(0), wid)
        _issue(x_hbm, buf_b, sem_b, jnp.int32(1), wid)

        def body(i, tops):
            g0 = i * 2
            tops = _process(x_hbm, out_hbm, buf_a, cm, outb, sem_a, g0, wid,
                            list(tops))
            _issue(x_hbm, buf_a, sem_a, g0 + 2, wid)
            tops = _process(x_hbm, out_hbm, buf_b, cm, outb, sem_b, g0 + 1,
                            wid, tops)
            _issue(x_hbm, buf_b, sem_b, g0 + 3, wid)
            return tuple(tops)

        init = tuple(jnp.full((L,), NEG, jnp.float32) for _ in range(K))
        lax.fori_loop(0, NG // 2, body, init)

    return run(inputs)


def kernel(inputs):
    out_tc = _topk_tc(inputs)
    out_sc = _topk_sc(inputs)
    return jnp.concatenate([out_tc, out_sc], axis=0)


# speedup vs baseline: 1.0760x; 1.0408x over previous
import functools

import jax
import jax.numpy as jnp
from jax import lax
from jax.experimental import pallas as pl
from jax.experimental.pallas import tpu as pltpu
from jax.experimental.pallas import tpu_sc as plsc

K = 8
S = 8192
C = 128
B = 64
BT = 12
BS = B - BT
L = 16
QROWS = 2048
NQ = S // QROWS
CH = 16
NCHUNK = QROWS // CH
NW = 32
TASKS_PER_W = (BS * (C // L)) // NW
NG = TASKS_PER_W * NQ
NEG = float("-inf")


def _tc_body(x_ref, o_ref):
    x = x_ref[0]
    iota = lax.broadcasted_iota(jnp.int32, (S, C), 0)
    neg = jnp.float32(NEG)
    for r in range(K):
        m = jnp.max(x, axis=0)
        o_ref[0, r, :] = m
        idx = jnp.min(jnp.where(x == m[None, :], iota, jnp.int32(S)), axis=0)
        x = jnp.where(iota == idx[None, :], neg, x)


def _topk_tc(inputs):
    return pl.pallas_call(
        _tc_body,
        grid=(BT,),
        in_specs=[pl.BlockSpec((1, S, C), lambda i: (i, 0, 0))],
        out_specs=pl.BlockSpec((1, K, C), lambda i: (i, 0, 0)),
        out_shape=jax.ShapeDtypeStruct((BT, K, C), jnp.float32),
    )(inputs)


def _bcast_bool(pred):
    return jnp.broadcast_to(pred, (L,))


def _insert_val(t, v):
    t = list(t)
    for l in range(K):
        gt = v > t[l]
        t[l], v = jnp.where(gt, v, t[l]), jnp.where(gt, t[l], v)
    return t


def _task_coords(g, wid):
    q = lax.rem(g, NQ)
    t = lax.div(g, NQ)
    tid = wid * TASKS_PER_W + t
    b = lax.div(tid, C // L)
    cb = lax.rem(tid, C // L)
    return b, cb, q


def _issue(x_hbm, buf, sem, g, wid):
    @pl.when(g < NG)
    def _():
        b, cb, q = _task_coords(g, wid)
        pltpu.async_copy(
            x_hbm.at[BT + b, pl.ds(q * QROWS, QROWS), pl.ds(cb * L, L)],
            buf, sem)


def _process(x_hbm, out_hbm, buf, cm, outb, sem, g, wid, tops):
    b, cb, q = _task_coords(g, wid)
    pltpu.make_async_copy(
        x_hbm.at[BT + b, pl.ds(q * QROWS, QROWS), pl.ds(cb * L, L)], buf,
        sem).wait()

    def p1_body(c, _):
        base = c * CH
        m = buf[base]
        for j in range(1, CH):
            m = jnp.maximum(m, buf[base + j])
        cm[c] = m
        return 0

    lax.fori_loop(0, NCHUNK, p1_body, 0)

    def p2_body(j, car):
        kv = list(car)
        kb = lax.bitcast_convert_type(cm[j], jnp.int32)
        mono = kb ^ ((kb >> 31) & jnp.int32(0x7FFFFFFF))
        key = (mono & jnp.int32(~(NCHUNK - 1))) | j
        for l in range(K):
            kv[l], key = jnp.maximum(kv[l], key), jnp.minimum(kv[l], key)
        return tuple(kv)

    init = tuple(jnp.full((L,), jnp.int32(-2**31), jnp.int32)
                 for _ in range(K))
    sel = lax.fori_loop(0, NCHUNK, p2_body, init, unroll=4)
    chunk_idx = [kv & jnp.int32(NCHUNK - 1) for kv in sel]

    fresh = _bcast_bool(q == 0)
    tops = [jnp.where(fresh, jnp.float32(NEG), tv) for tv in tops]

    lane = lax.broadcasted_iota(jnp.int32, (L,), 0)
    for r in range(K):
        base = chunk_idx[r] * CH

        def p3_body(j, car, base=base):
            rows = base + j
            gv = plsc.load_gather(buf, [rows, lane])
            return tuple(_insert_val(list(car), gv))

        tops = list(lax.fori_loop(0, CH, p3_body, tuple(tops), unroll=4))

    @pl.when(q == NQ - 1)
    def _():
        for r in range(K):
            outb[r] = tops[r]
        pltpu.sync_copy(outb, out_hbm.at[b, :, pl.ds(cb * L, L)])

    return tops


def _topk_sc(inputs):
    mesh = plsc.VectorSubcoreMesh(core_axis_name="c", subcore_axis_name="s")

    @functools.partial(
        pl.kernel,
        mesh=mesh,
        out_type=jax.ShapeDtypeStruct((BS, K, C), jnp.float32),
        compiler_params=pltpu.CompilerParams(use_tc_tiling_on_sc=False,
                                             needs_layout_passes=False),
        scratch_types=[
            pltpu.VMEM((QROWS, L), jnp.float32),
            pltpu.VMEM((QROWS, L), jnp.float32),
            pltpu.VMEM((NCHUNK, L), jnp.float32),
            pltpu.VMEM((K, L), jnp.float32),
            pltpu.SemaphoreType.DMA,
            pltpu.SemaphoreType.DMA,
        ],
    )
    def run(x_hbm, out_hbm, buf_a, buf_b, cm, outb, sem_a, sem_b):
        wid = lax.axis_index("s") * 2 + lax.axis_index("c")
        _issue(x_hbm, buf_a, sem_a, jnp.int32(0), wid)
        _issue(x_hbm, buf_b, sem_b, jnp.int32(1), wid)

        def body(i, tops):
            g0 = i * 2
            tops = _process(x_hbm, out_hbm, buf_a, cm, outb, sem_a, g0, wid,
                            list(tops))
            _issue(x_hbm, buf_a, sem_a, g0 + 2, wid)
            tops = _process(x_hbm, out_hbm, buf_b, cm, outb, sem_b, g0 + 1,
                            wid, tops)
            _issue(x_hbm, buf_b, sem_b, g0 + 3, wid)
            return tuple(tops)

        init = tuple(jnp.full((L,), NEG, jnp.float32) for _ in range(K))
        lax.fori_loop(0, NG // 2, body, init)

    return run(inputs)


def kernel(inputs):
    out_tc = _topk_tc(inputs)
    out_sc = _topk_sc(inputs)
    return jnp.concatenate([out_tc, out_sc], axis=0)


# speedup vs baseline: 1.1705x; 1.0878x over previous
import functools

import jax
import jax.numpy as jnp
from jax import lax
from jax.experimental import pallas as pl
from jax.experimental.pallas import tpu as pltpu
from jax.experimental.pallas import tpu_sc as plsc

K = 8
S = 8192
C = 128
B = 64
BT = 12
BS = B - BT
L = 16
QROWS = 2048
NQ = S // QROWS
CH = 16
NCHUNK = QROWS // CH
NW = 32
TASKS_PER_W = (BS * (C // L)) // NW
NG = TASKS_PER_W * NQ
NEG = float("-inf")


def _tc_body(x_ref, o_ref):
    x = x_ref[0]
    iota = lax.broadcasted_iota(jnp.int32, (S, C), 0)
    neg = jnp.float32(NEG)
    for r in range(K):
        m = jnp.max(x, axis=0)
        o_ref[0, r, :] = m
        idx = jnp.min(jnp.where(x == m[None, :], iota, jnp.int32(S)), axis=0)
        x = jnp.where(iota == idx[None, :], neg, x)


def _topk_tc(inputs):
    return pl.pallas_call(
        _tc_body,
        grid=(BT,),
        in_specs=[pl.BlockSpec((1, S, C), lambda i: (i, 0, 0))],
        out_specs=pl.BlockSpec((1, K, C), lambda i: (i, 0, 0)),
        out_shape=jax.ShapeDtypeStruct((BT, K, C), jnp.float32),
    )(inputs)


def _bcast_bool(pred):
    return jnp.broadcast_to(pred, (L,))


def _insert_val(t, v):
    t = list(t)
    for l in range(K):
        gt = v > t[l]
        t[l], v = jnp.where(gt, v, t[l]), jnp.where(gt, t[l], v)
    return t


def _task_coords(g, wid):
    q = lax.rem(g, NQ)
    t = lax.div(g, NQ)
    tid = wid * TASKS_PER_W + t
    b = lax.div(tid, C // L)
    cb = lax.rem(tid, C // L)
    return b, cb, q


def _issue(x_hbm, buf, sem, g, wid):
    @pl.when(g < NG)
    def _():
        b, cb, q = _task_coords(g, wid)
        pltpu.async_copy(
            x_hbm.at[BT + b, pl.ds(q * QROWS, QROWS), pl.ds(cb * L, L)],
            buf, sem)


def _process(x_hbm, out_hbm, buf, outb, sem, g, wid, tops):
    b, cb, q = _task_coords(g, wid)
    pltpu.make_async_copy(
        x_hbm.at[BT + b, pl.ds(q * QROWS, QROWS), pl.ds(cb * L, L)], buf,
        sem).wait()

    def p12_body(c, car):
        kv = list(car)
        base = c * CH
        m = buf[base]
        for j in range(1, CH):
            m = jnp.maximum(m, buf[base + j])
        kb = lax.bitcast_convert_type(m, jnp.int32)
        mono = kb ^ ((kb >> 31) & jnp.int32(0x7FFFFFFF))
        key = (mono & jnp.int32(~(NCHUNK - 1))) | c
        for l in range(K):
            kv[l], key = jnp.maximum(kv[l], key), jnp.minimum(kv[l], key)
        return tuple(kv)

    init = tuple(jnp.full((L,), jnp.int32(-2**31), jnp.int32)
                 for _ in range(K))
    sel = lax.fori_loop(0, NCHUNK, p12_body, init, unroll=2)
    chunk_idx = [kv & jnp.int32(NCHUNK - 1) for kv in sel]

    fresh = _bcast_bool(q == 0)
    tops = [jnp.where(fresh, jnp.float32(NEG), tv) for tv in tops]

    lane = lax.broadcasted_iota(jnp.int32, (L,), 0)
    for r in range(K):
        base = chunk_idx[r] * CH

        def p3_body(j, car, base=base):
            rows = base + j
            gv = plsc.load_gather(buf, [rows, lane])
            return tuple(_insert_val(list(car), gv))

        tops = list(lax.fori_loop(0, CH, p3_body, tuple(tops), unroll=4))

    @pl.when(q == NQ - 1)
    def _():
        for r in range(K):
            outb[r] = tops[r]
        pltpu.sync_copy(outb, out_hbm.at[b, :, pl.ds(cb * L, L)])

    return tops


def _topk_sc(inputs):
    mesh = plsc.VectorSubcoreMesh(core_axis_name="c", subcore_axis_name="s")

    @functools.partial(
        pl.kernel,
        mesh=mesh,
        out_type=jax.ShapeDtypeStruct((BS, K, C), jnp.float32),
        compiler_params=pltpu.CompilerParams(use_tc_tiling_on_sc=False,
                                             needs_layout_passes=False),
        scratch_types=[
            pltpu.VMEM((QROWS, L), jnp.float32),
            pltpu.VMEM((QROWS, L), jnp.float32),
            pltpu.VMEM((K, L), jnp.float32),
            pltpu.SemaphoreType.DMA,
            pltpu.SemaphoreType.DMA,
        ],
    )
    def run(x_hbm, out_hbm, buf_a, buf_b, outb, sem_a, sem_b):
        wid = lax.axis_index("s") * 2 + lax.axis_index("c")
        _issue(x_hbm, buf_a, sem_a, jnp.int32(0), wid)
        _issue(x_hbm, buf_b, sem_b, jnp.int32(1), wid)

        def body(i, tops):
            g0 = i * 2
            tops = _process(x_hbm, out_hbm, buf_a, outb, sem_a, g0, wid,
                            list(tops))
            _issue(x_hbm, buf_a, sem_a, g0 + 2, wid)
            tops = _process(x_hbm, out_hbm, buf_b, outb, sem_b, g0 + 1,
                            wid, tops)
            _issue(x_hbm, buf_b, sem_b, g0 + 3, wid)
            return tuple(tops)

        init = tuple(jnp.full((L,), NEG, jnp.float32) for _ in range(K))
        lax.fori_loop(0, NG // 2, body, init)

    return run(inputs)


def kernel(inputs):
    out_tc = _topk_tc(inputs)
    out_sc = _topk_sc(inputs)
    return jnp.concatenate([out_tc, out_sc], axis=0)


# speedup vs baseline: 1.2159x; 1.0388x over previous
import functools

import jax
import jax.numpy as jnp
from jax import lax
from jax.experimental import pallas as pl
from jax.experimental.pallas import tpu as pltpu
from jax.experimental.pallas import tpu_sc as plsc

K = 8
S = 8192
C = 128
B = 64
BT = 8
BS = B - BT
L = 16
QROWS = 2048
NQ = S // QROWS
CH = 16
NCHUNK = QROWS // CH
NW = 32
TASKS_PER_W = (BS * (C // L)) // NW
NG = TASKS_PER_W * NQ
NEG = float("-inf")


def _tc_body(x_ref, o_ref):
    x = x_ref[0]
    iota = lax.broadcasted_iota(jnp.int32, (S, C), 0)
    neg = jnp.float32(NEG)
    for r in range(K):
        m = jnp.max(x, axis=0)
        o_ref[0, r, :] = m
        idx = jnp.min(jnp.where(x == m[None, :], iota, jnp.int32(S)), axis=0)
        x = jnp.where(iota == idx[None, :], neg, x)


def _topk_tc(inputs):
    return pl.pallas_call(
        _tc_body,
        grid=(BT,),
        in_specs=[pl.BlockSpec((1, S, C), lambda i: (i, 0, 0))],
        out_specs=pl.BlockSpec((1, K, C), lambda i: (i, 0, 0)),
        out_shape=jax.ShapeDtypeStruct((BT, K, C), jnp.float32),
    )(inputs)


def _bcast_bool(pred):
    return jnp.broadcast_to(pred, (L,))


def _insert_val(t, v):
    t = list(t)
    for l in range(K):
        gt = v > t[l]
        t[l], v = jnp.where(gt, v, t[l]), jnp.where(gt, t[l], v)
    return t


def _task_coords(g, wid):
    q = lax.rem(g, NQ)
    t = lax.div(g, NQ)
    tid = wid * TASKS_PER_W + t
    b = lax.div(tid, C // L)
    cb = lax.rem(tid, C // L)
    return b, cb, q


def _issue(x_hbm, buf, sem, g, wid):
    @pl.when(g < NG)
    def _():
        b, cb, q = _task_coords(g, wid)
        pltpu.async_copy(
            x_hbm.at[BT + b, pl.ds(q * QROWS, QROWS), pl.ds(cb * L, L)],
            buf, sem)


def _process(x_hbm, out_hbm, buf, outb, sem, g, wid, tops):
    b, cb, q = _task_coords(g, wid)
    pltpu.make_async_copy(
        x_hbm.at[BT + b, pl.ds(q * QROWS, QROWS), pl.ds(cb * L, L)], buf,
        sem).wait()

    def p12_body(c, car):
        kv = list(car)
        base = c * CH
        m = buf[base]
        for j in range(1, CH):
            m = jnp.maximum(m, buf[base + j])
        kb = lax.bitcast_convert_type(m, jnp.int32)
        mono = kb ^ ((kb >> 31) & jnp.int32(0x7FFFFFFF))
        key = (mono & jnp.int32(~(NCHUNK - 1))) | c
        for l in range(K):
            kv[l], key = jnp.maximum(kv[l], key), jnp.minimum(kv[l], key)
        return tuple(kv)

    init = tuple(jnp.full((L,), jnp.int32(-2**31), jnp.int32)
                 for _ in range(K))
    sel = lax.fori_loop(0, NCHUNK, p12_body, init, unroll=2)
    chunk_idx = [kv & jnp.int32(NCHUNK - 1) for kv in sel]

    fresh = _bcast_bool(q == 0)
    tops = [jnp.where(fresh, jnp.float32(NEG), tv) for tv in tops]

    lane = lax.broadcasted_iota(jnp.int32, (L,), 0)
    for r in range(K):
        base = chunk_idx[r] * CH

        def p3_body(j, car, base=base):
            rows = base + j
            gv = plsc.load_gather(buf, [rows, lane])
            return tuple(_insert_val(list(car), gv))

        tops = list(lax.fori_loop(0, CH, p3_body, tuple(tops), unroll=4))

    @pl.when(q == NQ - 1)
    def _():
        for r in range(K):
            outb[r] = tops[r]
        pltpu.sync_copy(outb, out_hbm.at[b, :, pl.ds(cb * L, L)])

    return tops


def _topk_sc(inputs):
    mesh = plsc.VectorSubcoreMesh(core_axis_name="c", subcore_axis_name="s")

    @functools.partial(
        pl.kernel,
        mesh=mesh,
        out_type=jax.ShapeDtypeStruct((BS, K, C), jnp.float32),
        compiler_params=pltpu.CompilerParams(use_tc_tiling_on_sc=False,
                                             needs_layout_passes=False),
        scratch_types=[
            pltpu.VMEM((QROWS, L), jnp.float32),
            pltpu.VMEM((QROWS, L), jnp.float32),
            pltpu.VMEM((K, L), jnp.float32),
            pltpu.SemaphoreType.DMA,
            pltpu.SemaphoreType.DMA,
        ],
    )
    def run(x_hbm, out_hbm, buf_a, buf_b, outb, sem_a, sem_b):
        wid = lax.axis_index("s") * 2 + lax.axis_index("c")
        _issue(x_hbm, buf_a, sem_a, jnp.int32(0), wid)
        _issue(x_hbm, buf_b, sem_b, jnp.int32(1), wid)

        def body(i, tops):
            g0 = i * 2
            tops = _process(x_hbm, out_hbm, buf_a, outb, sem_a, g0, wid,
                            list(tops))
            _issue(x_hbm, buf_a, sem_a, g0 + 2, wid)
            tops = _process(x_hbm, out_hbm, buf_b, outb, sem_b, g0 + 1,
                            wid, tops)
            _issue(x_hbm, buf_b, sem_b, g0 + 3, wid)
            return tuple(tops)

        init = tuple(jnp.full((L,), NEG, jnp.float32) for _ in range(K))
        lax.fori_loop(0, NG // 2, body, init)

    return run(inputs)


def kernel(inputs):
    out_tc = _topk_tc(inputs)
    out_sc = _topk_sc(inputs)
    return jnp.concatenate([out_tc, out_sc], axis=0)
